# baseline (device time: 37922 ns/iter reference)
import jax
import jax.numpy as jnp
from jax import lax
from jax.experimental import pallas as pl
from jax.experimental.pallas import tpu as pltpu

C = 8


def kernel(x):
    _, m, n = x.shape
    half = n // 2
    rows_half = m // 2
    ck = rows_half // C

    def body(x_ref, out_ref, recv_y_ref, recv_x_ref, stage_ref, local_sem,
             stage_sems, ysend_sems, yrecv_sems, xsend_sems, xrecv_sems):
        my_x = lax.axis_index("x")
        my_y = lax.axis_index("y")
        my_z = lax.axis_index("z")
        peer = (my_x, 1 - my_y, my_z)
        xnbr = (1 - my_x, my_y, my_z)

        col0 = (1 - my_y) * half
        row0 = my_x * rows_half

        def stage_copy(c):
            return pltpu.make_async_copy(
                x_ref.at[0, pl.ds(row0 + c * ck, ck), pl.ds(col0, half)],
                stage_ref.at[pl.ds(c * ck, ck), :],
                stage_sems.at[c],
            )

        for c in range(C):
            stage_copy(c).start()

        mycol = my_y * half
        local_copy = pltpu.make_async_copy(
            x_ref.at[0, :, pl.ds(mycol, half)], out_ref, local_sem
        )
        local_copy.start()

        barrier_sem = pltpu.get_barrier_semaphore()
        for nbr in (peer, xnbr):
            pl.semaphore_signal(
                barrier_sem, inc=1,
                device_id=nbr, device_id_type=pl.DeviceIdType.MESH,
            )
        pl.semaphore_wait(barrier_sem, 2)

        def y_rdma(c):
            return pltpu.make_async_remote_copy(
                src_ref=stage_ref.at[pl.ds(c * ck, ck), :],
                dst_ref=recv_y_ref.at[pl.ds(c * ck, ck), :],
                send_sem=ysend_sems.at[c],
                recv_sem=yrecv_sems.at[c],
                device_id=peer,
                device_id_type=pl.DeviceIdType.MESH,
            )

        def x_rdma(c):
            return pltpu.make_async_remote_copy(
                src_ref=recv_y_ref.at[pl.ds(c * ck, ck), :],
                dst_ref=recv_x_ref.at[pl.ds(c * ck, ck), :],
                send_sem=xsend_sems.at[c],
                recv_sem=xrecv_sems.at[c],
                device_id=xnbr,
                device_id_type=pl.DeviceIdType.MESH,
            )

        for c in range(C):
            stage_copy(c).wait()
            y_rdma(c).start()

        local_copy.wait()
        other0 = (1 - my_x) * rows_half

        def add_direct(c):
            g = row0 + c * ck
            out_ref[pl.ds(g, ck), :] += recv_y_ref[pl.ds(c * ck, ck), :]

        def add_forwarded(c):
            g = other0 + c * ck
            out_ref[pl.ds(g, ck), :] += recv_x_ref[pl.ds(c * ck, ck), :]

        for c in range(C):
            y_rdma(c).wait_recv()
            x_rdma(c).start()
            add_direct(c)
            if c >= 1:
                x_rdma(c - 1).wait_recv()
                add_forwarded(c - 1)
        x_rdma(C - 1).wait_recv()
        add_forwarded(C - 1)

        for c in range(C):
            y_rdma(c).wait_send()
            x_rdma(c).wait_send()

    return pl.pallas_call(
        body,
        out_shape=jax.ShapeDtypeStruct((m, half), jnp.float32),
        in_specs=[pl.BlockSpec(memory_space=pltpu.VMEM)],
        out_specs=pl.BlockSpec(memory_space=pltpu.VMEM),
        scratch_shapes=[
            pltpu.VMEM((rows_half, half), jnp.float32),
            pltpu.VMEM((rows_half, half), jnp.float32),
            pltpu.VMEM((rows_half, half), jnp.float32),
            pltpu.SemaphoreType.DMA,
            pltpu.SemaphoreType.DMA((C,)),
            pltpu.SemaphoreType.DMA((C,)),
            pltpu.SemaphoreType.DMA((C,)),
            pltpu.SemaphoreType.DMA((C,)),
            pltpu.SemaphoreType.DMA((C,)),
        ],
        compiler_params=pltpu.CompilerParams(collective_id=0),
    )(x)


# device time: 32282 ns/iter; 1.1747x vs baseline; 1.1747x over previous
import jax
import jax.numpy as jnp
from jax import lax
from jax.experimental import pallas as pl
from jax.experimental.pallas import tpu as pltpu

C = 8


def kernel(x):
    _, m, n = x.shape
    half = n // 2
    Q = m // 4
    ck = Q // C

    def body(x_ref, out_ref,
             recv_y_ref, recv_x1_ref, recv_z1_ref, recv_x2_ref, recv_z2_ref,
             stage_ref, local_sem, stage_sems,
             ysend, yrecv, x1send, x1recv, z1send, z1recv,
             x2send, x2recv, z2send, z2recv):
        my_x = lax.axis_index("x")
        my_y = lax.axis_index("y")
        my_z = lax.axis_index("z")
        zbit = my_z % 2
        zp_z = my_z + 1 - 2 * zbit

        peer = (my_x, 1 - my_y, my_z)
        xp = (1 - my_x, my_y, my_z)
        zp = (my_x, my_y, zp_z)

        qid = 2 * my_x + zbit
        xp_qid = 2 * (1 - my_x) + zbit
        zp_qid = 2 * my_x + (1 - zbit)
        dg_qid = 2 * (1 - my_x) + (1 - zbit)

        col0 = (1 - my_y) * half
        qrow = qid * Q

        def stage_copy(c):
            return pltpu.make_async_copy(
                x_ref.at[0, pl.ds(qrow + c * ck, ck), pl.ds(col0, half)],
                stage_ref.at[pl.ds(c * ck, ck), :],
                stage_sems.at[c],
            )

        for c in range(C):
            stage_copy(c).start()

        mycol = my_y * half
        local_copy = pltpu.make_async_copy(
            x_ref.at[0, :, pl.ds(mycol, half)], out_ref, local_sem
        )
        local_copy.start()

        barrier_sem = pltpu.get_barrier_semaphore()
        for nbr in (peer, xp, zp):
            pl.semaphore_signal(
                barrier_sem, inc=1,
                device_id=nbr, device_id_type=pl.DeviceIdType.MESH,
            )
        pl.semaphore_wait(barrier_sem, 3)

        def rdma(src, dst, ssem, rsem, dev):
            return pltpu.make_async_remote_copy(
                src_ref=src, dst_ref=dst, send_sem=ssem, recv_sem=rsem,
                device_id=dev, device_id_type=pl.DeviceIdType.MESH,
            )

        def y_rdma(c):
            return rdma(stage_ref.at[pl.ds(c * ck, ck), :],
                        recv_y_ref.at[pl.ds(c * ck, ck), :],
                        ysend.at[c], yrecv.at[c], peer)

        def x1_rdma(c):
            return rdma(recv_y_ref.at[pl.ds(c * ck, ck), :],
                        recv_x1_ref.at[pl.ds(c * ck, ck), :],
                        x1send.at[c], x1recv.at[c], xp)

        def z1_rdma(c):
            return rdma(recv_y_ref.at[pl.ds(c * ck, ck), :],
                        recv_z1_ref.at[pl.ds(c * ck, ck), :],
                        z1send.at[c], z1recv.at[c], zp)

        def x2_rdma(c):
            k = c // 2
            return rdma(recv_z1_ref.at[pl.ds(c * ck, ck), :],
                        recv_x2_ref.at[pl.ds(k * ck, ck), :],
                        x2send.at[k], x2recv.at[k], xp)

        def z2_rdma(c):
            k = c // 2
            return rdma(recv_x1_ref.at[pl.ds(c * ck, ck), :],
                        recv_z2_ref.at[pl.ds(k * ck, ck), :],
                        z2send.at[k], z2recv.at[k], zp)

        for c in range(C):
            stage_copy(c).wait()
            y_rdma(c).start()

        local_copy.wait()

        def add(buf, slot, qidx, c):
            g = qidx * Q + c * ck
            out_ref[pl.ds(g, ck), :] += buf[pl.ds(slot * ck, ck), :]

        for c in range(C):
            y_rdma(c).wait_recv()
            x1_rdma(c).start()
            z1_rdma(c).start()
            add(recv_y_ref, c, qid, c)

        for c in range(C):
            x1_rdma(c).wait_recv()
            if c % 2 == 1:
                z2_rdma(c).start()
            add(recv_x1_ref, c, xp_qid, c)
            z1_rdma(c).wait_recv()
            if c % 2 == 0:
                x2_rdma(c).start()
            add(recv_z1_ref, c, zp_qid, c)

        for k in range(C // 2):
            x2_rdma(2 * k).wait_recv()
            add(recv_x2_ref, k, dg_qid, 2 * k)
            z2_rdma(2 * k + 1).wait_recv()
            add(recv_z2_ref, k, dg_qid, 2 * k + 1)

        for c in range(C):
            y_rdma(c).wait_send()
            x1_rdma(c).wait_send()
            z1_rdma(c).wait_send()
        for c in range(C // 2):
            x2_rdma(2 * c).wait_send()
            z2_rdma(2 * c + 1).wait_send()

    return pl.pallas_call(
        body,
        out_shape=jax.ShapeDtypeStruct((m, half), jnp.float32),
        in_specs=[pl.BlockSpec(memory_space=pltpu.VMEM)],
        out_specs=pl.BlockSpec(memory_space=pltpu.VMEM),
        scratch_shapes=[
            pltpu.VMEM((Q, half), jnp.float32),
            pltpu.VMEM((Q, half), jnp.float32),
            pltpu.VMEM((Q, half), jnp.float32),
            pltpu.VMEM((Q // 2, half), jnp.float32),
            pltpu.VMEM((Q // 2, half), jnp.float32),
            pltpu.VMEM((Q, half), jnp.float32),
            pltpu.SemaphoreType.DMA,
            pltpu.SemaphoreType.DMA((C,)),
            pltpu.SemaphoreType.DMA((C,)),
            pltpu.SemaphoreType.DMA((C,)),
            pltpu.SemaphoreType.DMA((C,)),
            pltpu.SemaphoreType.DMA((C,)),
            pltpu.SemaphoreType.DMA((C,)),
            pltpu.SemaphoreType.DMA((C,)),
            pltpu.SemaphoreType.DMA((C // 2,)),
            pltpu.SemaphoreType.DMA((C // 2,)),
            pltpu.SemaphoreType.DMA((C // 2,)),
            pltpu.SemaphoreType.DMA((C // 2,)),
        ],
        compiler_params=pltpu.CompilerParams(collective_id=0),
    )(x)


# device time: 29605 ns/iter; 1.2809x vs baseline; 1.0904x over previous
import jax
import jax.numpy as jnp
from jax import lax
from jax.experimental import pallas as pl
from jax.experimental.pallas import tpu as pltpu

C = 8
DY = (4, 5, 6, 7)
DX = (1, 3)
DZ = (0, 2)


def kernel(x):
    _, m, n = x.shape
    half = n // 2
    Q = m // 4
    ck = Q // C

    def body(x_ref, out_ref,
             recv_y_ref, recv_ye_ref, recv_x1_ref, recv_z1_ref,
             recv_x2_ref, recv_z2_ref, stage_ref, stage_e_ref,
             local_sem, stage_sems, stage_e_sems,
             ysend, yrecv, yesend, yerecv,
             x1send, x1recv, z1send, z1recv,
             x2send, x2recv, z2send, z2recv):
        my_x = lax.axis_index("x")
        my_y = lax.axis_index("y")
        my_z = lax.axis_index("z")
        zbit = my_z % 2
        zp_z = my_z + 1 - 2 * zbit

        peer = (my_x, 1 - my_y, my_z)
        xp = (1 - my_x, my_y, my_z)
        zp = (my_x, my_y, zp_z)

        qid = 2 * my_x + zbit
        xp_qid = 2 * (1 - my_x) + zbit
        zp_qid = 2 * my_x + (1 - zbit)
        dg_qid = 2 * (1 - my_x) + (1 - zbit)

        col0 = (1 - my_y) * half
        qrow = qid * Q
        dgrow = dg_qid * Q

        def stage_copy(c):
            return pltpu.make_async_copy(
                x_ref.at[0, pl.ds(qrow + c * ck, ck), pl.ds(col0, half)],
                stage_ref.at[pl.ds(c * ck, ck), :],
                stage_sems.at[c],
            )

        def stage_e_copy(k):
            return pltpu.make_async_copy(
                x_ref.at[0, pl.ds(dgrow + DY[k] * ck, ck), pl.ds(col0, half)],
                stage_e_ref.at[pl.ds(k * ck, ck), :],
                stage_e_sems.at[k],
            )

        for c in range(C):
            stage_copy(c).start()
        for k in range(len(DY)):
            stage_e_copy(k).start()

        mycol = my_y * half
        local_copy = pltpu.make_async_copy(
            x_ref.at[0, :, pl.ds(mycol, half)], out_ref, local_sem
        )
        local_copy.start()

        barrier_sem = pltpu.get_barrier_semaphore()
        for nbr in (peer, xp, zp):
            pl.semaphore_signal(
                barrier_sem, inc=1,
                device_id=nbr, device_id_type=pl.DeviceIdType.MESH,
            )
        pl.semaphore_wait(barrier_sem, 3)

        def rdma(src, dst, ssem, rsem, dev):
            return pltpu.make_async_remote_copy(
                src_ref=src, dst_ref=dst, send_sem=ssem, recv_sem=rsem,
                device_id=dev, device_id_type=pl.DeviceIdType.MESH,
            )

        def y_rdma(c):
            return rdma(stage_ref.at[pl.ds(c * ck, ck), :],
                        recv_y_ref.at[pl.ds(c * ck, ck), :],
                        ysend.at[c], yrecv.at[c], peer)

        def ye_rdma(k):
            return rdma(stage_e_ref.at[pl.ds(k * ck, ck), :],
                        recv_ye_ref.at[pl.ds(k * ck, ck), :],
                        yesend.at[k], yerecv.at[k], peer)

        def x1_rdma(c):
            return rdma(recv_y_ref.at[pl.ds(c * ck, ck), :],
                        recv_x1_ref.at[pl.ds(c * ck, ck), :],
                        x1send.at[c], x1recv.at[c], xp)

        def z1_rdma(c):
            return rdma(recv_y_ref.at[pl.ds(c * ck, ck), :],
                        recv_z1_ref.at[pl.ds(c * ck, ck), :],
                        z1send.at[c], z1recv.at[c], zp)

        def x2_rdma(k):
            return rdma(recv_z1_ref.at[pl.ds(DX[k] * ck, ck), :],
                        recv_x2_ref.at[pl.ds(k * ck, ck), :],
                        x2send.at[k], x2recv.at[k], xp)

        def z2_rdma(k):
            return rdma(recv_x1_ref.at[pl.ds(DZ[k] * ck, ck), :],
                        recv_z2_ref.at[pl.ds(k * ck, ck), :],
                        z2send.at[k], z2recv.at[k], zp)

        for c in range(C):
            stage_copy(c).wait()
            y_rdma(c).start()
        for k in range(len(DY)):
            stage_e_copy(k).wait()
            ye_rdma(k).start()

        local_copy.wait()

        def add(buf, slot, qidx, c):
            g = qidx * Q + c * ck
            out_ref[pl.ds(g, ck), :] += buf[pl.ds(slot * ck, ck), :]

        for c in range(C):
            y_rdma(c).wait_recv()
            x1_rdma(c).start()
            z1_rdma(c).start()
            add(recv_y_ref, c, qid, c)

        nx2 = nz2 = 0
        for c in range(C):
            x1_rdma(c).wait_recv()
            if c in DZ:
                z2_rdma(nz2).start()
                nz2 += 1
            add(recv_x1_ref, c, xp_qid, c)
            z1_rdma(c).wait_recv()
            if c in DX:
                x2_rdma(nx2).start()
                nx2 += 1
            add(recv_z1_ref, c, zp_qid, c)

        for k in range(len(DY)):
            ye_rdma(k).wait_recv()
            add(recv_ye_ref, k, dg_qid, DY[k])
        for k in range(len(DX)):
            x2_rdma(k).wait_recv()
            add(recv_x2_ref, k, dg_qid, DX[k])
        for k in range(len(DZ)):
            z2_rdma(k).wait_recv()
            add(recv_z2_ref, k, dg_qid, DZ[k])

        for c in range(C):
            y_rdma(c).wait_send()
            x1_rdma(c).wait_send()
            z1_rdma(c).wait_send()
        for k in range(len(DY)):
            ye_rdma(k).wait_send()
        for k in range(len(DX)):
            x2_rdma(k).wait_send()
        for k in range(len(DZ)):
            z2_rdma(k).wait_send()

    ndy, ndx, ndz = len(DY), len(DX), len(DZ)
    return pl.pallas_call(
        body,
        out_shape=jax.ShapeDtypeStruct((m, half), jnp.float32),
        in_specs=[pl.BlockSpec(memory_space=pltpu.VMEM)],
        out_specs=pl.BlockSpec(memory_space=pltpu.VMEM),
        scratch_shapes=[
            pltpu.VMEM((Q, half), jnp.float32),
            pltpu.VMEM((ndy * ck, half), jnp.float32),
            pltpu.VMEM((Q, half), jnp.float32),
            pltpu.VMEM((Q, half), jnp.float32),
            pltpu.VMEM((ndx * ck, half), jnp.float32),
            pltpu.VMEM((ndz * ck, half), jnp.float32),
            pltpu.VMEM((Q, half), jnp.float32),
            pltpu.VMEM((ndy * ck, half), jnp.float32),
            pltpu.SemaphoreType.DMA,
            pltpu.SemaphoreType.DMA((C,)),
            pltpu.SemaphoreType.DMA((ndy,)),
            pltpu.SemaphoreType.DMA((C,)),
            pltpu.SemaphoreType.DMA((C,)),
            pltpu.SemaphoreType.DMA((ndy,)),
            pltpu.SemaphoreType.DMA((ndy,)),
            pltpu.SemaphoreType.DMA((C,)),
            pltpu.SemaphoreType.DMA((C,)),
            pltpu.SemaphoreType.DMA((C,)),
            pltpu.SemaphoreType.DMA((C,)),
            pltpu.SemaphoreType.DMA((ndx,)),
            pltpu.SemaphoreType.DMA((ndx,)),
            pltpu.SemaphoreType.DMA((ndz,)),
            pltpu.SemaphoreType.DMA((ndz,)),
        ],
        compiler_params=pltpu.CompilerParams(collective_id=0),
    )(x)
